# baseline (speedup 1.0000x reference)
import functools

import jax
import jax.numpy as jnp
from jax import lax
from jax.experimental import pallas as pl
from jax.experimental.pallas import tpu as pltpu
from jax.experimental.pallas import tpu_sc as plsc

S, B, D = 4096, 4, 1024
N = S * B
NC, NS = 2, 16
NW = NC * NS
ROWS_PER_W = N // NW
SW = 4
W = SW * B
NB = 2
NITER = ROWS_PER_W // W
S_PER_W = S // NW


def _sc_gather_add(x, idx, pe):
  mesh = plsc.VectorSubcoreMesh(core_axis_name="c", subcore_axis_name="s")

  @functools.partial(
      pl.kernel,
      out_type=jax.ShapeDtypeStruct((S, B, D), jnp.float32),
      mesh=mesh,
      scratch_types=[
          pltpu.VMEM((ROWS_PER_W,), jnp.int32),
          pltpu.VMEM((NB, W, D), jnp.float32),
          pltpu.VMEM((NB, SW, B, D), jnp.float32),
      ] + [pltpu.SemaphoreType.DMA] * (3 * NB),
  )
  def k(x_hbm, idx_hbm, pe_hbm, out_hbm, idx_v, rows_v, x_v, *sems):
    gsem = sems[0:NB]
    xsem = sems[NB:2 * NB]
    osem = sems[2 * NB:3 * NB]
    wid = lax.axis_index("s") * NC + lax.axis_index("c")
    rbase = wid * ROWS_PER_W
    sbase = wid * S_PER_W
    pltpu.sync_copy(idx_hbm.at[pl.ds(rbase, ROWS_PER_W)], idx_v)

    def g_desc(i, b):
      return pltpu.make_async_copy(
          pe_hbm.at[idx_v.at[pl.ds(i * W, W)]], rows_v.at[b], gsem[b])

    def x_desc(i, b):
      return pltpu.make_async_copy(
          x_hbm.at[pl.ds(sbase + i * SW, SW)], x_v.at[b], xsem[b])

    def o_desc(i, b):
      return pltpu.make_async_copy(
          x_v.at[b], out_hbm.at[pl.ds(sbase + i * SW, SW)], osem[b])

    for b in range(NB):
      g_desc(b, b).start()
      x_desc(b, b).start()

    @pl.loop(0, NITER // NB)
    def _(step):
      for b in range(NB):
        i = step * NB + b
        g_desc(i, b).wait()
        x_desc(i, b).wait()

        @pl.loop(0, SW)
        def _(si):
          for bi in range(B):
            for c in range(0, D, 16):
              sl = pl.ds(c, 16)
              plsc.addupdate(x_v.at[b, si, bi, sl], rows_v[b, si * B + bi, sl])

        o_desc(i, b).start()

        @pl.when(i + NB < NITER)
        def _():
          g_desc(i + NB, b).start()

        bp = (b - 1) % NB
        ip = i - 1
        @pl.when((ip >= 0) & (ip + NB < NITER))
        def _():
          o_desc(ip, bp).wait()
          x_desc(ip + NB, bp).start()

    for i in range(NITER - NB, NITER):
      o_desc(i, i % NB).wait()

  return k(x, idx, pe)


def kernel(x, offset_order, pe):
  idx = offset_order.astype(jnp.int32).T.reshape(-1)
  return _sc_gather_add(x, idx, pe)


# speedup vs baseline: 1.0461x; 1.0461x over previous
import functools

import jax
import jax.numpy as jnp
from jax import lax
from jax.experimental import pallas as pl
from jax.experimental.pallas import tpu as pltpu
from jax.experimental.pallas import tpu_sc as plsc

S, B, D = 4096, 4, 1024
N = S * B
NC, NS = 2, 16
NW = NC * NS
ROWS_PER_W = N // NW
SW = 2
W = SW * B
NB = 4
NITER = ROWS_PER_W // W
S_PER_W = S // NW


def _sc_gather_add(x, idx, pe):
  mesh = plsc.VectorSubcoreMesh(core_axis_name="c", subcore_axis_name="s")

  @functools.partial(
      pl.kernel,
      out_type=jax.ShapeDtypeStruct((S, B, D), jnp.float32),
      mesh=mesh,
      scratch_types=[
          pltpu.VMEM((ROWS_PER_W,), jnp.int32),
          pltpu.VMEM((NB, W, D), jnp.float32),
          pltpu.VMEM((NB, SW, B, D), jnp.float32),
          pltpu.VMEM((NB, SW, B, D), jnp.float32),
      ] + [pltpu.SemaphoreType.DMA] * (3 * NB),
  )
  def k(x_hbm, idx_hbm, pe_hbm, out_hbm, idx_v, rows_v, x_v, o_v, *sems):
    gsem = sems[0:NB]
    xsem = sems[NB:2 * NB]
    osem = sems[2 * NB:3 * NB]
    wid = lax.axis_index("s") * NC + lax.axis_index("c")
    rbase = wid * ROWS_PER_W
    sbase = wid * S_PER_W
    pltpu.sync_copy(idx_hbm.at[pl.ds(rbase, ROWS_PER_W)], idx_v)

    def g_desc(i, b):
      return pltpu.make_async_copy(
          pe_hbm.at[idx_v.at[pl.ds(i * W, W)]], rows_v.at[b], gsem[b])

    def x_desc(i, b):
      return pltpu.make_async_copy(
          x_hbm.at[pl.ds(sbase + i * SW, SW)], x_v.at[b], xsem[b])

    def o_desc(i, b):
      return pltpu.make_async_copy(
          o_v.at[b], out_hbm.at[pl.ds(sbase + i * SW, SW)], osem[b])

    for b in range(NB):
      g_desc(b, b).start()
      x_desc(b, b).start()

    @pl.loop(0, NITER // NB)
    def _(step):
      for b in range(NB):
        i = step * NB + b
        g_desc(i, b).wait()
        x_desc(i, b).wait()

        @pl.when(i >= NB)
        def _():
          o_desc(i - NB, b).wait()

        @pl.loop(0, SW)
        def _(si):
          for bi in range(B):
            for c in range(0, D, 16):
              sl = pl.ds(c, 16)
              o_v[b, si, bi, sl] = rows_v[b, si * B + bi, sl] + x_v[b, si, bi, sl]

        o_desc(i, b).start()

        @pl.when(i + NB < NITER)
        def _():
          g_desc(i + NB, b).start()
          x_desc(i + NB, b).start()

    for b in range(NB):
      o_desc(NITER - NB + b, b).wait()

  return k(x, idx, pe)


def kernel(x, offset_order, pe):
  idx = offset_order.astype(jnp.int32).T.reshape(-1)
  return _sc_gather_add(x, idx, pe)


# speedup vs baseline: 1.0686x; 1.0216x over previous
import functools

import jax
import jax.numpy as jnp
from jax import lax
from jax.experimental import pallas as pl
from jax.experimental.pallas import tpu as pltpu
from jax.experimental.pallas import tpu_sc as plsc

S, B, D = 4096, 4, 1024
N = S * B
NC, NS = 2, 16
NW = NC * NS
ROWS_PER_W = N // NW
SW = 4
W = SW * B
NBG = 2
NBX = 4
NITER = ROWS_PER_W // W
S_PER_W = S // NW


def _sc_gather_add(x, idx, pe):
  mesh = plsc.VectorSubcoreMesh(core_axis_name="c", subcore_axis_name="s")

  @functools.partial(
      pl.kernel,
      out_type=jax.ShapeDtypeStruct((S, B, D), jnp.float32),
      mesh=mesh,
      scratch_types=[
          pltpu.VMEM((ROWS_PER_W,), jnp.int32),
          pltpu.VMEM((NBG, W, D), jnp.float32),
          pltpu.VMEM((NBX, SW, B, D), jnp.float32),
      ] + [pltpu.SemaphoreType.DMA] * (NBG + 2 * NBX),
  )
  def k(x_hbm, idx_hbm, pe_hbm, out_hbm, idx_v, rows_v, x_v, *sems):
    gsem = sems[0:NBG]
    xsem = sems[NBG:NBG + NBX]
    osem = sems[NBG + NBX:NBG + 2 * NBX]
    wid = lax.axis_index("s") * NC + lax.axis_index("c")
    rbase = wid * ROWS_PER_W
    sbase = wid * S_PER_W
    pltpu.sync_copy(idx_hbm.at[pl.ds(rbase, ROWS_PER_W)], idx_v)

    def g_desc(i, b):
      return pltpu.make_async_copy(
          pe_hbm.at[idx_v.at[pl.ds(i * W, W)]], rows_v.at[b], gsem[b])

    def x_desc(i, b):
      return pltpu.make_async_copy(
          x_hbm.at[pl.ds(sbase + i * SW, SW)], x_v.at[b], xsem[b])

    def o_desc(i, b):
      return pltpu.make_async_copy(
          x_v.at[b], out_hbm.at[pl.ds(sbase + i * SW, SW)], osem[b])

    for b in range(NBG):
      g_desc(b, b).start()
    for b in range(NBX):
      x_desc(b, b).start()

    @pl.loop(0, NITER // NBX)
    def _(step):
      for u in range(NBX):
        i = step * NBX + u
        bg = u % NBG
        bx = u
        g_desc(i, bg).wait()
        x_desc(i, bx).wait()

        @pl.loop(0, SW)
        def _(si):
          for bi in range(B):
            for c in range(0, D, 16):
              sl = pl.ds(c, 16)
              x_v[bx, si, bi, sl] = x_v[bx, si, bi, sl] + rows_v[bg, si * B + bi, sl]

        o_desc(i, bx).start()

        @pl.when(i + NBG < NITER)
        def _():
          g_desc(i + NBG, bg).start()

        bp = (u - 1) % NBX
        ip = i - 1
        @pl.when((ip >= 0) & (ip + NBX < NITER))
        def _():
          o_desc(ip, bp).wait()
          x_desc(ip + NBX, bp).start()

    for i in range(NITER - NBX, NITER):
      o_desc(i, i % NBX).wait()

  return k(x, idx, pe)


def kernel(x, offset_order, pe):
  idx = offset_order.astype(jnp.int32).T.reshape(-1)
  return _sc_gather_add(x, idx, pe)


# speedup vs baseline: 1.0975x; 1.0270x over previous
import functools

import jax
import jax.numpy as jnp
from jax import lax
from jax.experimental import pallas as pl
from jax.experimental.pallas import tpu as pltpu
from jax.experimental.pallas import tpu_sc as plsc

S, B, D = 4096, 4, 1024
N = S * B
NC, NS = 2, 16
NW = NC * NS
ROWS_PER_W = N // NW
SW = 4
W = SW * B
NB = 2
NITER = ROWS_PER_W // W
S_PER_W = S // NW


def _sc_gather_add(x, idx, pe):
  mesh = plsc.VectorSubcoreMesh(core_axis_name="c", subcore_axis_name="s")

  @functools.partial(
      pl.kernel,
      out_type=jax.ShapeDtypeStruct((S, B, D), jnp.float32),
      mesh=mesh,
      scratch_types=[
          pltpu.VMEM((ROWS_PER_W,), jnp.int32),
          pltpu.VMEM((NB, W, D), jnp.float32),
          pltpu.VMEM((NB, SW, B, D), jnp.float32),
          pltpu.VMEM((NB, SW, B, D), jnp.float32),
      ] + [pltpu.SemaphoreType.DMA] * (3 * NB),
  )
  def k(x_hbm, idx_hbm, pe_hbm, out_hbm, idx_v, rows_v, x_v, o_v, *sems):
    gsem = sems[0:NB]
    xsem = sems[NB:2 * NB]
    osem = sems[2 * NB:3 * NB]
    wid = lax.axis_index("s") * NC + lax.axis_index("c")
    rbase = wid * ROWS_PER_W
    sbase = wid * S_PER_W
    pltpu.sync_copy(idx_hbm.at[pl.ds(rbase, ROWS_PER_W)], idx_v)

    def g_desc(i, b):
      return pltpu.make_async_copy(
          pe_hbm.at[idx_v.at[pl.ds(i * W, W)]], rows_v.at[b], gsem[b])

    def x_desc(i, b):
      return pltpu.make_async_copy(
          x_hbm.at[pl.ds(sbase + i * SW, SW)], x_v.at[b], xsem[b])

    def o_desc(i, b):
      return pltpu.make_async_copy(
          o_v.at[b], out_hbm.at[pl.ds(sbase + i * SW, SW)], osem[b])

    for b in range(NB):
      g_desc(b, b).start()
      x_desc(b, b).start()

    @pl.loop(0, NITER // NB)
    def _(step):
      for b in range(NB):
        i = step * NB + b
        g_desc(i, b).wait()
        x_desc(i, b).wait()

        @pl.when(i >= NB)
        def _():
          o_desc(i - NB, b).wait()

        @pl.loop(0, SW)
        def _(si):
          for bi in range(B):
            for c in range(0, D, 16):
              sl = pl.ds(c, 16)
              o_v[b, si, bi, sl] = rows_v[b, si * B + bi, sl] + x_v[b, si, bi, sl]

        o_desc(i, b).start()

        @pl.when(i + NB < NITER)
        def _():
          g_desc(i + NB, b).start()
          x_desc(i + NB, b).start()

    for b in range(NB):
      o_desc(NITER - NB + b, b).wait()

  return k(x, idx, pe)


def kernel(x, offset_order, pe):
  idx = offset_order.astype(jnp.int32).T.reshape(-1)
  return _sc_gather_add(x, idx, pe)
